# baseline (speedup 1.0000x reference)
import functools

import jax
import jax.numpy as jnp
from jax import lax
from jax.experimental import pallas as pl
from jax.experimental.pallas import tpu as pltpu
from jax.experimental.pallas import tpu_sc as plsc

_NC = 2
_NS = 16
_NW = _NC * _NS
_B = 16384
_D = 64
_BPW = _B // _NW
_NSLOT = 16
_EPS = 1e-6


def _vsqrt(x):
    xs = jnp.maximum(x, jnp.float32(1e-35))
    i = lax.bitcast_convert_type(xs, jnp.int32)
    i = jnp.int32(0x5F3759DF) - lax.shift_right_logical(i, 1)
    y = lax.bitcast_convert_type(i, jnp.float32)
    for _ in range(3):
        y = y * (jnp.float32(1.5) - jnp.float32(0.5) * xs * y * y)
    return xs * y


_mesh = plsc.VectorSubcoreMesh(core_axis_name="c", subcore_axis_name="s")


@functools.partial(
    pl.kernel,
    out_type=jax.ShapeDtypeStruct((_NW, 16), jnp.float32),
    mesh=_mesh,
    compiler_params=pltpu.CompilerParams(needs_layout_passes=False),
    scratch_types=[
        pltpu.VMEM((_BPW,), jnp.int32),
        pltpu.VMEM((2 * _NSLOT * 8, _D), jnp.float32),
        pltpu.VMEM((_BPW, _D), jnp.float32),
        pltpu.VMEM((16,), jnp.float32),
        [pltpu.SemaphoreType.DMA] * _NSLOT,
        pltpu.SemaphoreType.DMA,
    ],
)
def _center_loss_sc(table, labels, batch, out, lab_v, ring_v, batch_v,
                    acc_v, sems, semb):
    wid = lax.axis_index("s") * _NC + lax.axis_index("c")

    cpb = pltpu.async_copy(batch.at[pl.ds(wid * _BPW, _BPW)], batch_v, semb)
    pltpu.sync_copy(labels.at[pl.ds(wid * _BPW, _BPW)], lab_v)
    cpb.wait()

    lanes = lax.iota(jnp.int32, 16)

    def fire(labv, i, bank, slot):
        gid = lax.shift_right_logical(labv[i], 3)
        s = bank * _NSLOT + slot
        return pltpu.async_copy(
            table.at[gid], ring_v.at[pl.ds(s * 8, 8)], sems[slot])

    def block_labels(blk):
        return lab_v[pl.ds(jnp.minimum(blk, _BPW // 16 - 1) * 16, 16)]

    def drain(bank):
        for i in range(16):
            s = bank * _NSLOT + i
            pltpu.make_async_copy(
                table.at[0], ring_v.at[pl.ds(s * 8, 8)], sems[i]).wait()

    def fire_bank(blk, bank):
        labv = block_labels(blk)
        for i in range(16):
            fire(labv, i, bank, i)

    def compute_block(blk, bank, acc):
        labv = block_labels(blk)
        sub = jnp.bitwise_and(labv, 7)
        rows_g = jnp.int32(bank * _NSLOT * 8) + lanes * 8 + sub
        rows_b = blk * 16 + lanes
        tot = jnp.zeros((16,), jnp.float32)
        for f in range(_D):
            fv = jnp.full((16,), f, jnp.int32)
            g = plsc.load_gather(ring_v, [rows_g, fv])
            b = plsc.load_gather(batch_v, [rows_b, fv])
            d = g - b + jnp.float32(_EPS)
            tot = tot + d * d
        return acc + _vsqrt(tot)

    fire_bank(jnp.int32(0), 0)

    def pair_body(it, acc):
        blk = 2 * it
        drain(0)
        fire_bank(blk + 1, 1)
        acc = compute_block(blk, 0, acc)
        drain(1)
        fire_bank(blk + 2, 0)
        acc = compute_block(blk + 1, 1, acc)
        return acc

    acc = lax.fori_loop(0, _BPW // 32, pair_body,
                        jnp.zeros((16,), jnp.float32))
    drain(0)

    acc_v[...] = acc
    pltpu.sync_copy(acc_v, out.at[wid])


def kernel(center_list, batch_center_vecs, gt_labels):
    table3 = center_list.reshape(125000, 8, _D)
    partials = _center_loss_sc(table3, gt_labels, batch_center_vecs)
    return jnp.sum(partials) / jnp.float32(_B)


# speedup vs baseline: 1.1028x; 1.1028x over previous
import functools

import jax
import jax.numpy as jnp
from jax import lax
from jax.experimental import pallas as pl
from jax.experimental.pallas import tpu as pltpu
from jax.experimental.pallas import tpu_sc as plsc

_NC = 2
_NS = 16
_NW = _NC * _NS
_B = 16384
_D = 64
_BPW = _B // _NW
_NSLOT = 16
_EPS = 1e-6


def _vsqrt(x):
    xs = jnp.maximum(x, jnp.float32(1e-35))
    i = lax.bitcast_convert_type(xs, jnp.int32)
    i = jnp.int32(0x5F3759DF) - lax.shift_right_logical(i, 1)
    y = lax.bitcast_convert_type(i, jnp.float32)
    for _ in range(3):
        y = y * (jnp.float32(1.5) - jnp.float32(0.5) * xs * y * y)
    return xs * y


_mesh = plsc.VectorSubcoreMesh(core_axis_name="c", subcore_axis_name="s")


@functools.partial(
    pl.kernel,
    out_type=jax.ShapeDtypeStruct((_NW, 16), jnp.float32),
    mesh=_mesh,
    compiler_params=pltpu.CompilerParams(needs_layout_passes=False),
    scratch_types=[
        pltpu.VMEM((_BPW,), jnp.int32),
        pltpu.VMEM((4, 16, 8, _D), jnp.float32),
        pltpu.VMEM((4, 16, _D), jnp.float32),
        pltpu.VMEM((16,), jnp.float32),
        [pltpu.SemaphoreType.DMA] * 4,
    ],
)
def _center_loss_sc(table, labels, batch, out, lab_v, ring_v, batch_v,
                    acc_v, sems):
    wid = lax.axis_index("s") * _NC + lax.axis_index("c")

    pltpu.sync_copy(labels.at[pl.ds(wid * _BPW, _BPW)], lab_v)

    lanes = lax.iota(jnp.int32, 16)

    def block_labels(blk):
        return lab_v[pl.ds(jnp.minimum(blk, _BPW // 16 - 1) * 16, 16)]

    def fire_bank(blk, bank):
        labv = block_labels(blk)
        for i in range(16):
            gid = lax.shift_right_logical(labv[i], 3)
            pltpu.async_copy(table.at[gid], ring_v.at[bank, i], sems[bank])
        grp = wid * (_BPW // 16) + jnp.minimum(blk, _BPW // 16 - 1)
        pltpu.async_copy(batch.at[grp], batch_v.at[bank], sems[bank])

    def drain_bank(bank):
        pltpu.make_async_copy(
            table.at[pl.ds(0, 16)], ring_v.at[bank], sems[bank]).wait()
        pltpu.make_async_copy(
            batch.at[0], batch_v.at[bank], sems[bank]).wait()

    def compute_block(blk, bank, acc):
        labv = block_labels(blk)
        sub = jnp.bitwise_and(labv, 7)
        bankv = jnp.full((16,), bank, jnp.int32)
        tot = jnp.zeros((16,), jnp.float32)
        for f in range(_D):
            fv = jnp.full((16,), f, jnp.int32)
            g = plsc.load_gather(ring_v, [bankv, lanes, sub, fv])
            b = plsc.load_gather(batch_v, [bankv, lanes, fv])
            d = g - b + jnp.float32(_EPS)
            tot = tot + d * d
        return acc + _vsqrt(tot)

    fire_bank(jnp.int32(0), 0)
    fire_bank(jnp.int32(1), 1)

    def quad_body(it, acc):
        blk = 4 * it
        for q in range(4):
            drain_bank(q)
            fire_bank(blk + q + 2, (q + 2) % 4)
            acc = compute_block(blk + q, q, acc)
        return acc

    acc = lax.fori_loop(0, _BPW // 64, quad_body,
                        jnp.zeros((16,), jnp.float32))
    drain_bank(0)
    drain_bank(1)

    acc_v[...] = acc
    pltpu.sync_copy(acc_v, out.at[wid])


def kernel(center_list, batch_center_vecs, gt_labels):
    table3 = center_list.reshape(125000, 8, _D)
    batch3 = batch_center_vecs.reshape(_B // 16, 16, _D)
    partials = _center_loss_sc(table3, gt_labels, batch3)
    return jnp.sum(partials) / jnp.float32(_B)


# speedup vs baseline: 1.1031x; 1.0003x over previous
import functools

import jax
import jax.numpy as jnp
from jax import lax
from jax.experimental import pallas as pl
from jax.experimental.pallas import tpu as pltpu
from jax.experimental.pallas import tpu_sc as plsc

_NC = 2
_NS = 16
_NW = _NC * _NS
_B = 16384
_D = 64
_BPW = _B // _NW
_NSLOT = 16
_EPS = 1e-6


def _vsqrt(x):
    xs = jnp.maximum(x, jnp.float32(1e-35))
    i = lax.bitcast_convert_type(xs, jnp.int32)
    i = jnp.int32(0x5F3759DF) - lax.shift_right_logical(i, 1)
    y = lax.bitcast_convert_type(i, jnp.float32)
    for _ in range(3):
        y = y * (jnp.float32(1.5) - jnp.float32(0.5) * xs * y * y)
    return xs * y


_mesh = plsc.VectorSubcoreMesh(core_axis_name="c", subcore_axis_name="s")


@functools.partial(
    pl.kernel,
    out_type=jax.ShapeDtypeStruct((_NW, 16), jnp.float32),
    mesh=_mesh,
    compiler_params=pltpu.CompilerParams(needs_layout_passes=False),
    scratch_types=[
        pltpu.VMEM((_BPW,), jnp.int32),
        pltpu.VMEM((4 * 16 * 8, _D), jnp.float32),
        pltpu.VMEM((4 * 16, _D), jnp.float32),
        pltpu.VMEM((16,), jnp.float32),
        [pltpu.SemaphoreType.DMA] * 4,
    ],
)
def _center_loss_sc(table, labels, batch, out, lab_v, ring_v,
                    batch_v, acc_v, sems):
    wid = lax.axis_index("s") * _NC + lax.axis_index("c")

    pltpu.sync_copy(labels.at[pl.ds(wid * _BPW, _BPW)], lab_v)

    lanes = lax.iota(jnp.int32, 16)

    def block_labels(blk):
        return lab_v[pl.ds(jnp.minimum(blk, _BPW // 16 - 1) * 16, 16)]

    def fire_bank(blk, bank):
        labv = block_labels(blk)
        for i in range(16):
            gid = lax.shift_right_logical(labv[i], 3)
            pltpu.async_copy(
                table.at[gid],
                ring_v.at[pl.ds((bank * 16 + i) * 8, 8)], sems[bank])
        grp = wid * (_BPW // 16) + jnp.minimum(blk, _BPW // 16 - 1)
        pltpu.async_copy(
            batch.at[grp], batch_v.at[pl.ds(bank * 16, 16)], sems[bank])

    def drain_bank(bank):
        pltpu.make_async_copy(
            table.at[pl.ds(0, 16)],
            ring_v.at[pl.ds(bank * 128, 128)],
            sems[bank]).wait()
        pltpu.make_async_copy(
            batch.at[0], batch_v.at[pl.ds(bank * 16, 16)], sems[bank]).wait()

    def compute_block(blk, bank, acc):
        labv = block_labels(blk)
        sub = jnp.bitwise_and(labv, 7)
        rows_g = jnp.int32(bank * 128) + lanes * 8 + sub
        rows_b = jnp.int32(bank * 16) + lanes
        tot = jnp.zeros((16,), jnp.float32)
        for f in range(_D):
            fv = jnp.full((16,), f, jnp.int32)
            g = plsc.load_gather(ring_v, [rows_g, fv])
            b = plsc.load_gather(batch_v, [rows_b, fv])
            d = g - b + jnp.float32(_EPS)
            tot = tot + d * d
        return acc + _vsqrt(tot)

    fire_bank(jnp.int32(0), 0)
    fire_bank(jnp.int32(1), 1)

    def quad_body(it, acc):
        blk = 4 * it
        for q in range(4):
            drain_bank(q)
            fire_bank(blk + q + 2, (q + 2) % 4)
            acc = compute_block(blk + q, q, acc)
        return acc

    acc = lax.fori_loop(0, _BPW // 64, quad_body,
                        jnp.zeros((16,), jnp.float32))
    drain_bank(0)
    drain_bank(1)

    acc_v[...] = acc
    pltpu.sync_copy(acc_v, out.at[wid])


def kernel(center_list, batch_center_vecs, gt_labels):
    table3 = center_list.reshape(125000, 8, _D)
    batch3 = batch_center_vecs.reshape(_B // 16, 16, _D)
    partials = _center_loss_sc(table3, gt_labels, batch3)
    return jnp.sum(partials) / jnp.float32(_B)
